# baseline (speedup 1.0000x reference)
import jax
import jax.numpy as jnp
from jax import lax
from jax.experimental import pallas as pl
from jax.experimental.pallas import tpu as pltpu
from jax.experimental.pallas import tpu_sc as plsc

_N_USERS = 25000
_N_NODES = 50000
_NP = 50048
_DIM = 64
_HALF = 32
_N_EDGES = 800000
_N_HOPS = 3

_NC = 2
_NS = 16

_BUF_E = 256
_SUP_E = 1024
_NSUP = 50
_NPAIR = _NSUP // 2
_EDGES_PER_TILE = _NSUP * _SUP_E
_E_PAD = _NS * _EDGES_PER_TILE
_E_ALLOC = _E_PAD + _SUP_E

_ROWS_PER_TILE = _NP // _NS
_RCHUNK = 184
_NRC = _ROWS_PER_TILE // _RCHUNK


def _sc_body(agg0, col1, row1, val1, out,
             cidx_a, ridx_a, vals_a, cidx_b, ridx_b, vals_b, gb0, gb1, msg,
             acc, gs0, gs1, ssem, ia0, ia1, ia2, ib0, ib1, ib2):
    c = lax.axis_index("c")
    s = lax.axis_index("s")

    node0 = pl.multiple_of(s * _ROWS_PER_TILE, 8)
    eoff0 = s * _EDGES_PER_TILE
    cbase = pl.multiple_of(c * _NP, 8)
    obase = pl.multiple_of(c * (4 * _NP), 8)

    gbufs = (gb0, gb1)
    gsems = (gs0, gs1)

    def _cp0(j, _):
        n0 = pl.multiple_of(node0 + j * _RCHUNK, 8)
        pltpu.sync_copy(agg0.at[pl.ds(cbase + n0, _RCHUNK), :],
                        msg.at[pl.ds(0, _RCHUNK), :])
        pltpu.sync_copy(msg.at[pl.ds(0, _RCHUNK), :],
                        out.at[pl.ds(obase + n0, _RCHUNK), :])
        return _
    lax.fori_loop(0, _NRC, _cp0, None)

    def _zrow(i, _):
        msg[i, pl.ds(0, 16)] = jnp.zeros((16,), jnp.float32)
        msg[i, pl.ds(16, 16)] = jnp.zeros((16,), jnp.float32)
        return _

    def _hop(hop, _):
        lax.fori_loop(0, _RCHUNK, _zrow, None)

        def _zcp(j, _):
            n0 = pl.multiple_of(node0 + j * _RCHUNK, 8)
            pltpu.async_copy(msg.at[pl.ds(0, _RCHUNK), :],
                             acc.at[pl.ds(n0, _RCHUNK), :], ssem)
            return _
        lax.fori_loop(0, _NRC, _zcp, None)

        def _zwait(j, _):
            n0 = pl.multiple_of(node0 + j * _RCHUNK, 8)
            pltpu.make_async_copy(msg.at[pl.ds(0, _RCHUNK), :],
                                  acc.at[pl.ds(n0, _RCHUNK), :], ssem).wait()
            return _
        lax.fori_loop(0, _NRC, _zwait, None)
        plsc.subcore_barrier()

        src_off = obase + hop * _NP

        def _fire_idx(eoff, cv, rv, vv, s0, s1, s2):
            pltpu.async_copy(col1.at[pl.ds(eoff, _SUP_E)], cv, s0)
            pltpu.async_copy(row1.at[pl.ds(eoff, _SUP_E)], rv, s1)
            pltpu.async_copy(val1.at[pl.ds(eoff, _SUP_E)],
                             vv.at[pl.ds(0, _SUP_E)], s2)

        def _wait_idx(eoff, cv, rv, vv, s0, s1, s2):
            pltpu.make_async_copy(col1.at[pl.ds(eoff, _SUP_E)], cv, s0).wait()
            pltpu.make_async_copy(row1.at[pl.ds(eoff, _SUP_E)], rv, s1).wait()
            pltpu.make_async_copy(val1.at[pl.ds(eoff, _SUP_E)],
                                  vv.at[pl.ds(0, _SUP_E)], s2).wait()

        def _pipeline(cidx_v, ridx_v, vals_v):
            def _ci(j):
                return cidx_v.at[pl.ds(j * _BUF_E, _BUF_E)]

            def _ri(j):
                return ridx_v.at[pl.ds(j * _BUF_E, _BUF_E)]

            nj = _SUP_E // _BUF_E
            gd = [None, None]
            sd = None
            gd[0] = pltpu.async_copy(out.at[_ci(0)], gbufs[0], gsems[0])
            for j in range(nj):
                b = j % 2
                nb = (j + 1) % 2
                if j + 1 < nj:
                    gd[nb] = pltpu.async_copy(
                        out.at[_ci(j + 1)], gbufs[nb], gsems[nb])
                gd[b].wait()
                if sd is not None:
                    sd.wait()
                gbuf = gbufs[b]

                def _grp(g, _):
                    v16 = vals_v[pl.ds(j * _BUF_E + g * 16, 16)]
                    for l in range(16):
                        e = g * 16 + l
                        vl = v16[l]
                        msg[e, pl.ds(0, 16)] = gbuf[e, pl.ds(0, 16)] * vl
                        msg[e, pl.ds(16, 16)] = gbuf[e, pl.ds(16, 16)] * vl
                    return _
                lax.fori_loop(0, _BUF_E // 16, _grp, None)
                sd = pltpu.async_copy(msg, acc.at[_ri(j)], ssem, add=True)
            sd.wait()

        def _adjust(cidx_v):
            for t in range(_SUP_E // 16):
                sl = pl.ds(t * 16, 16)
                cidx_v[sl] = cidx_v[sl] + src_off

        _fire_idx(pl.multiple_of(eoff0, 512),
                  cidx_a, ridx_a, vals_a, ia0, ia1, ia2)

        def _pair(i, _):
            eA = pl.multiple_of(eoff0 + (2 * i) * _SUP_E, 512)
            eB = pl.multiple_of(eA + _SUP_E, 512)
            eN = pl.multiple_of(eB + _SUP_E, 512)
            _fire_idx(eB, cidx_b, ridx_b, vals_b, ib0, ib1, ib2)
            _wait_idx(eA, cidx_a, ridx_a, vals_a, ia0, ia1, ia2)
            _adjust(cidx_a)
            _pipeline(cidx_a, ridx_a, vals_a)
            _fire_idx(eN, cidx_a, ridx_a, vals_a, ia0, ia1, ia2)
            _wait_idx(eB, cidx_b, ridx_b, vals_b, ib0, ib1, ib2)
            _adjust(cidx_b)
            _pipeline(cidx_b, ridx_b, vals_b)
            return _
        lax.fori_loop(0, _NPAIR, _pair, None)
        _wait_idx(pl.multiple_of(eoff0 + _NSUP * _SUP_E, 512),
                  cidx_a, ridx_a, vals_a, ia0, ia1, ia2)
        plsc.subcore_barrier()

        def _rd(j, b):
            n0 = pl.multiple_of(node0 + j * _RCHUNK, 8)
            return pltpu.async_copy(acc.at[pl.ds(n0, _RCHUNK), :],
                                    gbufs[b].at[pl.ds(0, _RCHUNK), :],
                                    gsems[b])

        def _wr(j, b, sem):
            n0 = pl.multiple_of(node0 + j * _RCHUNK, 8)
            return pltpu.async_copy(gbufs[b].at[pl.ds(0, _RCHUNK), :],
                                    out.at[pl.ds(src_off + _NP + n0,
                                                 _RCHUNK), :], sem)

        wsems = (ia0, ia1)
        rd = [None, None]
        wr = [None, None]
        rd[0] = _rd(0, 0)
        for j in range(_NRC):
            b = j % 2
            nb = (j + 1) % 2
            if j + 1 < _NRC:
                if wr[nb] is not None:
                    wr[nb].wait()
                rd[nb] = _rd(j + 1, nb)
            rd[b].wait()
            wr[b] = _wr(j, b, wsems[b])
        wr[(_NRC - 1) % 2].wait()
        wr[_NRC % 2].wait()
        plsc.subcore_barrier()
        return _

    lax.fori_loop(0, _N_HOPS, _hop, None)


@jax.jit
def _propagate(agg0, col1, row1, val1):
    mesh = plsc.VectorSubcoreMesh(core_axis_name="c", subcore_axis_name="s")
    f = pl.kernel(
        _sc_body,
        out_type=jax.ShapeDtypeStruct((_NC * (_N_HOPS + 1) * _NP, _HALF),
                                      jnp.float32),
        mesh=mesh,
        scratch_types=[
            pltpu.VMEM((_SUP_E,), jnp.int32),
            pltpu.VMEM((_SUP_E,), jnp.int32),
            pltpu.VMEM((_SUP_E + 16,), jnp.float32),
            pltpu.VMEM((_SUP_E,), jnp.int32),
            pltpu.VMEM((_SUP_E,), jnp.int32),
            pltpu.VMEM((_SUP_E + 16,), jnp.float32),
            pltpu.VMEM((_BUF_E, _HALF), jnp.float32),
            pltpu.VMEM((_BUF_E, _HALF), jnp.float32),
            pltpu.VMEM((_BUF_E, _HALF), jnp.float32),
            pltpu.VMEM_SHARED((_NP, _HALF), jnp.float32),
        ] + [pltpu.SemaphoreType.DMA] * 9,
        compiler_params=pltpu.CompilerParams(use_tc_tiling_on_sc=False),
    )
    return f(agg0, col1, row1, val1)


def kernel(user_embed, item_embed, adj_indices, adj_values):
    all_embed = jnp.concatenate([user_embed, item_embed], axis=0)
    npad = _NP - _N_NODES
    agg0 = jnp.concatenate([
        jnp.pad(all_embed[:, :_HALF], ((0, npad), (0, 0))),
        jnp.pad(all_embed[:, _HALF:], ((0, npad), (0, 0))),
    ], axis=0)
    pad = _E_ALLOC - _N_EDGES
    row = jnp.pad(adj_indices[0], (0, pad))
    col = jnp.pad(adj_indices[1], (0, pad))
    val = jnp.pad(adj_values, (0, pad))
    flat = _propagate(agg0, col, row, val)
    embs = (flat.reshape(_NC, _N_HOPS + 1, _NP, _HALF)
            .transpose(2, 1, 0, 3)
            .reshape(_NP, _N_HOPS + 1, _DIM)[:_N_NODES])
    return (embs[:_N_USERS], embs[_N_USERS:], embs)


# speedup vs baseline: 1.3899x; 1.3899x over previous
import jax
import jax.numpy as jnp
from jax import lax
from jax.experimental import pallas as pl
from jax.experimental.pallas import tpu as pltpu
from jax.experimental.pallas import tpu_sc as plsc

_N_USERS = 25000
_N_NODES = 50000
_NP = 50048
_DIM = 64
_HALF = 32
_N_EDGES = 800000
_N_HOPS = 3

_NC = 2
_NS = 16

_BUF_E = 256
_SUP_E = 1024
_NSUP = 49
_EDGES_PER_TILE = _NSUP * _SUP_E
_E_PAD = _NS * _EDGES_PER_TILE
_E_ALLOC = _E_PAD

_ROWS_PER_TILE = _NP // _NS
_RCHUNK = 184
_NRC = _ROWS_PER_TILE // _RCHUNK


def _sc_body(agg0, col1, row1, val1, out,
             cidx_a, ridx_a, vals_a, cidx_b, ridx_b, vals_b, gb0, gb1, msg,
             acc, gs0, gs1, ssem, ia0, ia1, ia2, ib0, ib1, ib2):
    c = lax.axis_index("c")
    s = lax.axis_index("s")

    node0 = pl.multiple_of(s * _ROWS_PER_TILE, 8)
    eoff0 = s * _EDGES_PER_TILE
    cbase = pl.multiple_of(c * _NP, 8)
    obase = pl.multiple_of(c * (4 * _NP), 8)

    gbufs = (gb0, gb1)
    gsems = (gs0, gs1)

    def _cp0(j, _):
        n0 = pl.multiple_of(node0 + j * _RCHUNK, 8)
        pltpu.sync_copy(agg0.at[pl.ds(cbase + n0, _RCHUNK), :],
                        msg.at[pl.ds(0, _RCHUNK), :])
        pltpu.sync_copy(msg.at[pl.ds(0, _RCHUNK), :],
                        out.at[pl.ds(obase + n0, _RCHUNK), :])
        return _
    lax.fori_loop(0, _NRC, _cp0, None)

    def _zrow(i, _):
        msg[i, pl.ds(0, 16)] = jnp.zeros((16,), jnp.float32)
        msg[i, pl.ds(16, 16)] = jnp.zeros((16,), jnp.float32)
        return _

    def _hop(hop, _):
        lax.fori_loop(0, _RCHUNK, _zrow, None)

        def _zcp(j, _):
            n0 = pl.multiple_of(node0 + j * _RCHUNK, 8)
            pltpu.async_copy(msg.at[pl.ds(0, _RCHUNK), :],
                             acc.at[pl.ds(n0, _RCHUNK), :], ssem)
            return _
        lax.fori_loop(0, _NRC, _zcp, None)

        def _zwait(j, _):
            n0 = pl.multiple_of(node0 + j * _RCHUNK, 8)
            pltpu.make_async_copy(msg.at[pl.ds(0, _RCHUNK), :],
                                  acc.at[pl.ds(n0, _RCHUNK), :], ssem).wait()
            return _
        lax.fori_loop(0, _NRC, _zwait, None)
        plsc.subcore_barrier()

        src_off = obase + hop * _NP

        def _fire_idx(eoff, cv, rv, vv, s0, s1, s2):
            pltpu.async_copy(col1.at[pl.ds(eoff, _SUP_E)], cv, s0)
            pltpu.async_copy(row1.at[pl.ds(eoff, _SUP_E)], rv, s1)
            pltpu.async_copy(val1.at[pl.ds(eoff, _SUP_E)],
                             vv.at[pl.ds(0, _SUP_E)], s2)

        def _wait_idx(eoff, cv, rv, vv, s0, s1, s2):
            pltpu.make_async_copy(col1.at[pl.ds(eoff, _SUP_E)], cv, s0).wait()
            pltpu.make_async_copy(row1.at[pl.ds(eoff, _SUP_E)], rv, s1).wait()
            pltpu.make_async_copy(val1.at[pl.ds(eoff, _SUP_E)],
                                  vv.at[pl.ds(0, _SUP_E)], s2).wait()

        def _pipeline(cidx_v, ridx_v, vals_v):
            def _ci(j):
                return cidx_v.at[pl.ds(j * _BUF_E, _BUF_E)]

            def _ri(j):
                return ridx_v.at[pl.ds(j * _BUF_E, _BUF_E)]

            nj = _SUP_E // _BUF_E
            gd = [None, None]
            sd = None
            gd[0] = pltpu.async_copy(out.at[_ci(0)], gbufs[0], gsems[0])
            for j in range(nj):
                b = j % 2
                nb = (j + 1) % 2
                if j + 1 < nj:
                    gd[nb] = pltpu.async_copy(
                        out.at[_ci(j + 1)], gbufs[nb], gsems[nb])
                gd[b].wait()
                if sd is not None:
                    sd.wait()
                gbuf = gbufs[b]

                def _grp(g, _):
                    v16 = vals_v[pl.ds(j * _BUF_E + g * 16, 16)]
                    for l in range(16):
                        e = g * 16 + l
                        vl = v16[l]
                        msg[e, pl.ds(0, 16)] = gbuf[e, pl.ds(0, 16)] * vl
                        msg[e, pl.ds(16, 16)] = gbuf[e, pl.ds(16, 16)] * vl
                    return _
                lax.fori_loop(0, _BUF_E // 16, _grp, None)
                sd = pltpu.async_copy(msg, acc.at[_ri(j)], ssem, add=True)
            sd.wait()

        def _adjust(cidx_v):
            for t in range(_SUP_E // 16):
                sl = pl.ds(t * 16, 16)
                cidx_v[sl] = cidx_v[sl] + src_off

        def _super(sidx, _):
            eoff = pl.multiple_of(eoff0 + sidx * _SUP_E, 512)
            _fire_idx(eoff, cidx_a, ridx_a, vals_a, ia0, ia1, ia2)
            _wait_idx(eoff, cidx_a, ridx_a, vals_a, ia0, ia1, ia2)
            _adjust(cidx_a)
            _pipeline(cidx_a, ridx_a, vals_a)
            return _
        lax.fori_loop(0, _NSUP, _super, None)
        plsc.subcore_barrier()

        def _rd(j, b):
            n0 = pl.multiple_of(node0 + j * _RCHUNK, 8)
            return pltpu.async_copy(acc.at[pl.ds(n0, _RCHUNK), :],
                                    gbufs[b].at[pl.ds(0, _RCHUNK), :],
                                    gsems[b])

        def _wr(j, b, sem):
            n0 = pl.multiple_of(node0 + j * _RCHUNK, 8)
            return pltpu.async_copy(gbufs[b].at[pl.ds(0, _RCHUNK), :],
                                    out.at[pl.ds(src_off + _NP + n0,
                                                 _RCHUNK), :], sem)

        wsems = (ia0, ia1)
        rd = [None, None]
        wr = [None, None]
        rd[0] = _rd(0, 0)
        for j in range(_NRC):
            b = j % 2
            nb = (j + 1) % 2
            if j + 1 < _NRC:
                if wr[nb] is not None:
                    wr[nb].wait()
                rd[nb] = _rd(j + 1, nb)
            rd[b].wait()
            wr[b] = _wr(j, b, wsems[b])
        wr[(_NRC - 1) % 2].wait()
        wr[_NRC % 2].wait()
        plsc.subcore_barrier()
        return _

    lax.fori_loop(0, _N_HOPS, _hop, None)


@jax.jit
def _propagate(agg0, col1, row1, val1):
    mesh = plsc.VectorSubcoreMesh(core_axis_name="c", subcore_axis_name="s")
    f = pl.kernel(
        _sc_body,
        out_type=jax.ShapeDtypeStruct((_NC * (_N_HOPS + 1) * _NP, _HALF),
                                      jnp.float32),
        mesh=mesh,
        scratch_types=[
            pltpu.VMEM((_SUP_E,), jnp.int32),
            pltpu.VMEM((_SUP_E,), jnp.int32),
            pltpu.VMEM((_SUP_E + 16,), jnp.float32),
            pltpu.VMEM((_SUP_E,), jnp.int32),
            pltpu.VMEM((_SUP_E,), jnp.int32),
            pltpu.VMEM((_SUP_E + 16,), jnp.float32),
            pltpu.VMEM((_BUF_E, _HALF), jnp.float32),
            pltpu.VMEM((_BUF_E, _HALF), jnp.float32),
            pltpu.VMEM((_BUF_E, _HALF), jnp.float32),
            pltpu.VMEM_SHARED((_NP, _HALF), jnp.float32),
        ] + [pltpu.SemaphoreType.DMA] * 9,
        compiler_params=pltpu.CompilerParams(use_tc_tiling_on_sc=False),
    )
    return f(agg0, col1, row1, val1)


def kernel(user_embed, item_embed, adj_indices, adj_values):
    all_embed = jnp.concatenate([user_embed, item_embed], axis=0)
    npad = _NP - _N_NODES
    agg0 = jnp.concatenate([
        jnp.pad(all_embed[:, :_HALF], ((0, npad), (0, 0))),
        jnp.pad(all_embed[:, _HALF:], ((0, npad), (0, 0))),
    ], axis=0)
    pad = _E_ALLOC - _N_EDGES
    row = jnp.pad(adj_indices[0], (0, pad))
    col = jnp.pad(adj_indices[1], (0, pad))
    val = jnp.pad(adj_values, (0, pad))
    flat = _propagate(agg0, col, row, val)
    embs = (flat.reshape(_NC, _N_HOPS + 1, _NP, _HALF)
            .transpose(2, 1, 0, 3)
            .reshape(_NP, _N_HOPS + 1, _DIM)[:_N_NODES])
    return (embs[:_N_USERS], embs[_N_USERS:], embs)
